# baseline (speedup 1.0000x reference)
import functools

import jax
import jax.numpy as jnp
from jax import lax
from jax.experimental import pallas as pl
from jax.experimental.pallas import tpu as pltpu
from jax.experimental.pallas import tpu_sc as plsc

N = 100000
E = 1600000
G = 128
IN_DIM = 32
HID = 146
HPAD = 160
NCHUNK = HPAD // 16
NCLS = 10
L = 4
NBLK = 800

NSUB = 16
EPT = 101376
E_PAD = EPT * NSUB
ROWS_PER_TILE = EPT // 128
SB = 3
NWAY = 4
NTRIP = ROWS_PER_TILE // (NWAY * SB)
NPAD_SH = 100096
SH_PER_TILE = NPAD_SH // NSUB
ZROWS = 391
NZCOPY = SH_PER_TILE // ZROWS

_sc_mesh = plsc.VectorSubcoreMesh(core_axis_name="c", subcore_axis_name="s")
_sc_params = pltpu.CompilerParams(use_tc_tiling_on_sc=False)


def _zero_fill(ref, nrows):
    def body(i, _):
        ref[i] = jnp.zeros((16,), jnp.float32)
        return 0

    lax.fori_loop(0, nrows, body, 0)


def _sc_degree_body(srcm, dstm, out_o, out_i, cnt_sh, ones_v, i1, zbuf):
    cid = lax.axis_index("c")
    sid = lax.axis_index("s")
    _zero_fill(zbuf, ZROWS)

    def fill_ones(i, _):
        ones_v[i] = jnp.ones((16,), jnp.float32)
        return 0

    lax.fori_loop(0, 128, fill_ones, 0)

    z0 = sid * SH_PER_TILE

    def zc(t, _):
        pltpu.sync_copy(zbuf, cnt_sh.at[pl.ds(z0 + t * ZROWS, ZROWS)])
        return 0

    lax.fori_loop(0, NZCOPY, zc, 0)
    plsc.subcore_barrier()

    row0 = sid * ROWS_PER_TILE

    for half in range(2):
        @pl.when(cid == half)
        def _():
            idxm = srcm if half == 0 else dstm

            def body(r, _):
                pltpu.sync_copy(idxm.at[pl.ds(row0 + r, 1)], i1)
                pltpu.sync_copy(ones_v, cnt_sh.at[i1.at[0]], add=True)
                return 0

            lax.fori_loop(0, ROWS_PER_TILE, body, 0)

    plsc.subcore_barrier()
    for half in range(2):
        @pl.when(cid == half)
        def _():
            out = out_o if half == 0 else out_i
            pltpu.sync_copy(cnt_sh.at[pl.ds(z0, SH_PER_TILE)],
                            out.at[pl.ds(z0, SH_PER_TILE)])


def _sc_degrees(srcm_deg, dstm):
    return pl.kernel(
        _sc_degree_body,
        out_type=[
            jax.ShapeDtypeStruct((NPAD_SH, 16), jnp.float32),
            jax.ShapeDtypeStruct((NPAD_SH, 16), jnp.float32),
        ],
        mesh=_sc_mesh,
        compiler_params=_sc_params,
        scratch_types=[
            pltpu.VMEM_SHARED((NPAD_SH, 16), jnp.float32),
            pltpu.VMEM((128, 16), jnp.float32),
            pltpu.VMEM((1, 128), jnp.int32),
            pltpu.VMEM((ZROWS, 16), jnp.float32),
        ],
    )(srcm_deg, dstm)


def _sc_agg_body(*refs):
    xs = refs[0:NCHUNK]
    eidx = refs[NCHUNK]
    zhbm = refs[NCHUNK + 1]
    dummy = refs[NCHUNK + 2]
    dummy_i = refs[NCHUNK + 3]
    ys = refs[NCHUNK + 4:2 * NCHUNK + 4]
    sc = refs[2 * NCHUNK + 4:]
    agg_sh = sc[0]
    rows = sc[1:1 + NWAY]
    idx = sc[1 + NWAY:1 + 2 * NWAY]
    semG = sc[1 + 2 * NWAY:1 + 3 * NWAY]
    semS = sc[1 + 3 * NWAY:1 + 4 * NWAY]
    semI = sc[1 + 4 * NWAY:1 + 5 * NWAY]

    cid = lax.axis_index("c")
    sid = lax.axis_index("s")
    z0 = sid * SH_PER_TILE
    row0 = sid * ROWS_PER_TILE

    for half in range(2):
        @pl.when(cid == half)
        def _():
            for c in range(half * 5, half * 5 + 5):
                table = xs[c]
                out = ys[c]
                pltpu.sync_copy(zhbm, agg_sh.at[pl.ds(z0, SH_PER_TILE)])
                plsc.subcore_barrier()

                for p in range(NWAY):
                    pltpu.sync_copy(eidx.at[pl.ds(row0 + p * SB, SB)], idx[p])

                def quad(bb, _):
                    gs = [None] * NWAY
                    for p in range(NWAY):
                        @pl.when(bb > 0)
                        def _wait_idx():
                            pltpu.make_async_copy(dummy_i, idx[p],
                                                  semI[p]).wait()
                        gs[p] = [
                            pltpu.async_copy(table.at[idx[p].at[j, 0]],
                                             rows[p].at[j], semG[p])
                            for j in range(SB)
                        ]
                    for p in range(NWAY):
                        for cp in gs[p]:
                            cp.wait()
                        for j in range(SB):
                            pltpu.async_copy(rows[p].at[j],
                                             agg_sh.at[idx[p].at[j, 1]],
                                             semS[p], add=True)
                    for p in range(NWAY):
                        pltpu.make_async_copy(dummy, rows[p], semS[p]).wait()

                        @pl.when(bb < NTRIP - 1)
                        def _prefetch():
                            base = row0 + ((bb + 1) * NWAY + p) * SB
                            pltpu.async_copy(eidx.at[pl.ds(base, SB)],
                                             idx[p], semI[p])
                    return 0

                lax.fori_loop(0, NTRIP, quad, 0)
                plsc.subcore_barrier()
                pltpu.sync_copy(agg_sh.at[pl.ds(z0, SH_PER_TILE)],
                                out.at[pl.ds(z0, SH_PER_TILE)])
                plsc.subcore_barrier()


def _sc_aggregate(xs, eidx, zhbm, dummy, dummy_i):
    return pl.kernel(
        _sc_agg_body,
        out_type=[jax.ShapeDtypeStruct((NPAD_SH, 16), jnp.float32)
                  for _ in range(NCHUNK)],
        mesh=_sc_mesh,
        compiler_params=_sc_params,
        scratch_types=(
            [pltpu.VMEM_SHARED((NPAD_SH, 16), jnp.float32)]
            + [pltpu.VMEM((SB, 128, 16), jnp.float32) for _ in range(NWAY)]
            + [pltpu.VMEM((SB, 2, 128), jnp.int32) for _ in range(NWAY)]
            + [pltpu.SemaphoreType.DMA for _ in range(3 * NWAY)]
        ),
    )(*xs, eidx, zhbm, dummy, dummy_i)


def _emb_body(nf_ref, w_ref, b_ref, w0_ref, no_ref, h_ref, *out_refs):
    h = (jnp.dot(nf_ref[...], w_ref[...],
                 preferred_element_type=jnp.float32)
         + b_ref[...][0:1, :])
    h_ref[...] = h
    acc = jnp.dot(h, w0_ref[...],
                  preferred_element_type=jnp.float32) * no_ref[...]
    for c in range(NCHUNK):
        out_refs[c][...] = acc[:, 16 * c:16 * (c + 1)]


def _emb_mm(nf, wp, bp, w0p, no2d):
    return pl.pallas_call(
        _emb_body,
        grid=(N // NBLK,),
        in_specs=[
            pl.BlockSpec((NBLK, IN_DIM), lambda i: (i, 0)),
            pl.BlockSpec((IN_DIM, HPAD), lambda i: (0, 0)),
            pl.BlockSpec((8, HPAD), lambda i: (0, 0)),
            pl.BlockSpec((HPAD, HPAD), lambda i: (0, 0)),
            pl.BlockSpec((NBLK, 1), lambda i: (i, 0)),
        ],
        out_specs=([pl.BlockSpec((NBLK, HPAD), lambda i: (i, 0))]
                   + [pl.BlockSpec((NBLK, 16), lambda i: (i, 0))
                      for _ in range(NCHUNK)]),
        out_shape=([jax.ShapeDtypeStruct((N, HPAD), jnp.float32)]
                   + [jax.ShapeDtypeStruct((N, 16), jnp.float32)
                      for _ in range(NCHUNK)]),
    )(nf, wp, bp, w0p, no2d)


def _layer_mm_body(h_ref, w_ref, no_ref, *out_refs):
    acc = jnp.dot(h_ref[...], w_ref[...],
                  preferred_element_type=jnp.float32) * no_ref[...]
    for c in range(NCHUNK):
        out_refs[c][...] = acc[:, 16 * c:16 * (c + 1)]


def _layer_mm(h, wp, no2d):
    return pl.pallas_call(
        _layer_mm_body,
        grid=(N // NBLK,),
        in_specs=[
            pl.BlockSpec((NBLK, HPAD), lambda i: (i, 0)),
            pl.BlockSpec((HPAD, HPAD), lambda i: (0, 0)),
            pl.BlockSpec((NBLK, 1), lambda i: (i, 0)),
        ],
        out_specs=[pl.BlockSpec((NBLK, 16), lambda i: (i, 0))
                   for _ in range(NCHUNK)],
        out_shape=[jax.ShapeDtypeStruct((N, 16), jnp.float32)
                   for _ in range(NCHUNK)],
    )(h, wp, no2d)


def _stats_body(*refs):
    ys = refs[0:NCHUNK]
    ni_ref, nn_ref, b_ref = refs[NCHUNK:NCHUNK + 3]
    s1_ref, s2_ref = refs[NCHUNK + 3:]

    @pl.when(pl.program_id(0) == 0)
    def _init():
        s1_ref[...] = jnp.zeros_like(s1_ref)
        s2_ref[...] = jnp.zeros_like(s2_ref)

    ni = ni_ref[...]
    nn = nn_ref[...]
    for c in range(NCHUNK):
        h2 = (ys[c][...] * ni + b_ref[...][0:1, 16 * c:16 * (c + 1)]) * nn
        s1_ref[0:1, 16 * c:16 * (c + 1)] += jnp.sum(h2, axis=0, keepdims=True)
        s2_ref[0:1, 16 * c:16 * (c + 1)] += jnp.sum(h2 * h2, axis=0,
                                                    keepdims=True)


def _stats(ys, ni2d, nn2d, bvec):
    return pl.pallas_call(
        _stats_body,
        grid=(N // NBLK,),
        in_specs=(
            [pl.BlockSpec((NBLK, 16), lambda i: (i, 0))
             for _ in range(NCHUNK)]
            + [pl.BlockSpec((NBLK, 1), lambda i: (i, 0)),
               pl.BlockSpec((NBLK, 1), lambda i: (i, 0)),
               pl.BlockSpec((8, HPAD), lambda i: (0, 0))]
        ),
        out_specs=[pl.BlockSpec((8, HPAD), lambda i: (0, 0)),
                   pl.BlockSpec((8, HPAD), lambda i: (0, 0))],
        out_shape=[jax.ShapeDtypeStruct((8, HPAD), jnp.float32),
                   jax.ShapeDtypeStruct((8, HPAD), jnp.float32)],
    )(*ys, ni2d, nn2d, bvec)


def _apply_mm_body(*refs):
    ys = refs[0:NCHUNK]
    (ni_ref, nn_ref, b_ref, hin_ref, s1_ref, s2_ref, g_ref, be_ref,
     w_ref, no_ref) = refs[NCHUNK:NCHUNK + 10]
    hout_ref = refs[NCHUNK + 10]
    out_refs = refs[NCHUNK + 11:]
    ni = ni_ref[...]
    nn = nn_ref[...]
    mu = s1_ref[...] * (1.0 / N)
    var = s2_ref[...] * (1.0 / N) - mu * mu
    scv = jax.lax.rsqrt(var + 1e-5) * g_ref[...]
    shv = be_ref[...] - mu * scv
    cols = []
    for c in range(NCHUNK):
        sl = slice(16 * c, 16 * (c + 1))
        h2 = (ys[c][...] * ni + b_ref[...][0:1, sl]) * nn
        v = h2 * scv[0:1, sl] + shv[0:1, sl]
        cols.append(hin_ref[...][:, sl] + jnp.maximum(v, 0.0))
    h = jnp.concatenate(cols, axis=1)
    hout_ref[...] = h
    acc = jnp.dot(h, w_ref[...],
                  preferred_element_type=jnp.float32) * no_ref[...]
    for c in range(NCHUNK):
        out_refs[c][...] = acc[:, 16 * c:16 * (c + 1)]


def _apply_mm(ys, ni2d, nn2d, bvec, h_in, s1, s2, gvec, bevec, wp, no2d):
    return pl.pallas_call(
        _apply_mm_body,
        grid=(N // NBLK,),
        in_specs=(
            [pl.BlockSpec((NBLK, 16), lambda i: (i, 0))
             for _ in range(NCHUNK)]
            + [pl.BlockSpec((NBLK, 1), lambda i: (i, 0)),
               pl.BlockSpec((NBLK, 1), lambda i: (i, 0)),
               pl.BlockSpec((8, HPAD), lambda i: (0, 0)),
               pl.BlockSpec((NBLK, HPAD), lambda i: (i, 0)),
               pl.BlockSpec((8, HPAD), lambda i: (0, 0)),
               pl.BlockSpec((8, HPAD), lambda i: (0, 0)),
               pl.BlockSpec((8, HPAD), lambda i: (0, 0)),
               pl.BlockSpec((8, HPAD), lambda i: (0, 0)),
               pl.BlockSpec((HPAD, HPAD), lambda i: (0, 0)),
               pl.BlockSpec((NBLK, 1), lambda i: (i, 0))]
        ),
        out_specs=([pl.BlockSpec((NBLK, HPAD), lambda i: (i, 0))]
                   + [pl.BlockSpec((NBLK, 16), lambda i: (i, 0))
                      for _ in range(NCHUNK)]),
        out_shape=([jax.ShapeDtypeStruct((N, HPAD), jnp.float32)]
                   + [jax.ShapeDtypeStruct((N, 16), jnp.float32)
                      for _ in range(NCHUNK)]),
    )(*ys, ni2d, nn2d, bvec, h_in, s1, s2, gvec, bevec, wp, no2d)


def _apply_readout_body(*refs):
    ys = refs[0:NCHUNK]
    (ni_ref, nn_ref, b_ref, hin_ref, s1_ref, s2_ref, g_ref, be_ref,
     gid_ref) = refs[NCHUNK:NCHUNK + 9]
    sums_ref, cnt_ref = refs[NCHUNK + 9:]

    @pl.when(pl.program_id(0) == 0)
    def _init():
        sums_ref[...] = jnp.zeros_like(sums_ref)
        cnt_ref[...] = jnp.zeros_like(cnt_ref)

    ni = ni_ref[...]
    nn = nn_ref[...]
    mu = s1_ref[...] * (1.0 / N)
    var = s2_ref[...] * (1.0 / N) - mu * mu
    scv = jax.lax.rsqrt(var + 1e-5) * g_ref[...]
    shv = be_ref[...] - mu * scv
    cols = []
    for c in range(NCHUNK):
        sl = slice(16 * c, 16 * (c + 1))
        h2 = (ys[c][...] * ni + b_ref[...][0:1, sl]) * nn
        v = h2 * scv[0:1, sl] + shv[0:1, sl]
        cols.append(hin_ref[...][:, sl] + jnp.maximum(v, 0.0))
    h = jnp.concatenate(cols, axis=1)
    gid = gid_ref[...]
    onehot = (gid == jax.lax.broadcasted_iota(jnp.int32, (NBLK, G), 1)).astype(
        jnp.float32
    )
    sums_ref[...] += jnp.dot(onehot.T, h, preferred_element_type=jnp.float32)
    cnt_ref[...] += jnp.dot(
        onehot.T, jnp.ones((NBLK, 8), jnp.float32),
        preferred_element_type=jnp.float32
    )


def _apply_readout(ys, ni2d, nn2d, bvec, h_in, s1, s2, gvec, bevec, gid2d):
    return pl.pallas_call(
        _apply_readout_body,
        grid=(N // NBLK,),
        in_specs=(
            [pl.BlockSpec((NBLK, 16), lambda i: (i, 0))
             for _ in range(NCHUNK)]
            + [pl.BlockSpec((NBLK, 1), lambda i: (i, 0)),
               pl.BlockSpec((NBLK, 1), lambda i: (i, 0)),
               pl.BlockSpec((8, HPAD), lambda i: (0, 0)),
               pl.BlockSpec((NBLK, HPAD), lambda i: (i, 0)),
               pl.BlockSpec((8, HPAD), lambda i: (0, 0)),
               pl.BlockSpec((8, HPAD), lambda i: (0, 0)),
               pl.BlockSpec((8, HPAD), lambda i: (0, 0)),
               pl.BlockSpec((8, HPAD), lambda i: (0, 0)),
               pl.BlockSpec((NBLK, 1), lambda i: (i, 0))]
        ),
        out_specs=[pl.BlockSpec((G, HPAD), lambda i: (0, 0)),
                   pl.BlockSpec((G, 8), lambda i: (0, 0))],
        out_shape=[jax.ShapeDtypeStruct((G, HPAD), jnp.float32),
                   jax.ShapeDtypeStruct((G, 8), jnp.float32)],
    )(*ys, ni2d, nn2d, bvec, h_in, s1, s2, gvec, bevec, gid2d)


def _mlp_body(sums_ref, cnt_ref, w1_ref, b1_ref, w2_ref, b2_ref, w3_ref, b3_ref,
              out_ref):
    cnt = jnp.maximum(cnt_ref[...][:, 0:1], 1.0)
    hg = sums_ref[...] / cnt
    z = jnp.maximum(jnp.dot(hg, w1_ref[...], preferred_element_type=jnp.float32)
                    + b1_ref[...][0:1, :], 0.0)
    z = jnp.maximum(jnp.dot(z, w2_ref[...], preferred_element_type=jnp.float32)
                    + b2_ref[...][0:1, :], 0.0)
    out_ref[...] = (jnp.dot(z, w3_ref[...], preferred_element_type=jnp.float32)
                    + b3_ref[...][0:1, :])


def _mlp(sums, cnt, w1p, b1p, w2p, b2p, w3p, b3p):
    return pl.pallas_call(
        _mlp_body,
        out_shape=jax.ShapeDtypeStruct((G, 128), jnp.float32),
    )(sums, cnt, w1p, b1p, w2p, b2p, w3p, b3p)


def _pad2(a, r, c):
    return jnp.pad(a, ((0, r - a.shape[0]), (0, c - a.shape[1])))


def kernel(nodes_feat, edges_feat, nodes_num_norm_sqrt, edges_num_norm_sqrt,
           edge_index, graph_ids, emb_W, emb_b, Ws, bs, gammas, betas,
           W1, b1, W2, b2, W3, b3):
    src = edge_index[0]
    dst = edge_index[1]
    epad = E_PAD - E
    srcm_agg = jnp.concatenate(
        [src, jnp.zeros((epad,), jnp.int32)]).reshape(-1, 128)
    dstm_agg = jnp.concatenate(
        [dst, jnp.full((epad,), N, jnp.int32)]).reshape(-1, 128)
    srcm_deg = jnp.concatenate(
        [src, jnp.full((epad,), N, jnp.int32)]).reshape(-1, 128)
    dstm = jnp.concatenate(
        [dst, jnp.full((epad,), N, jnp.int32)]).reshape(-1, 128)

    zhbm = jnp.zeros((SH_PER_TILE, 16), jnp.float32)
    dummy = jnp.zeros((SB, 128, 16), jnp.float32)
    dummy_i = jnp.zeros((SB, 2, 128), jnp.int32)
    eidx = jnp.stack([srcm_agg, dstm_agg], axis=1)
    dcnt_o, dcnt_i = _sc_degrees(srcm_deg, dstm)
    no2d = jnp.clip(dcnt_o[:N, 0:1], 1.0, None) ** -0.5
    ni2d = jnp.clip(dcnt_i[:N, 0:1], 1.0, None) ** -0.5
    nn2d = nodes_num_norm_sqrt

    embWp = jnp.pad(emb_W, ((0, 0), (0, HPAD - HID)))
    embbp = jnp.broadcast_to(jnp.pad(emb_b, (0, HPAD - HID)), (8, HPAD))
    wps = [jnp.pad(Ws[l], ((0, HPAD - HID), (0, HPAD - HID)))
           for l in range(L)]
    gid2d = graph_ids.reshape(N, 1)
    h, *xs = _emb_mm(nodes_feat, embWp, embbp, wps[0], no2d)
    for l in range(L):
        h_in = h
        bvec = jnp.broadcast_to(jnp.pad(bs[l], (0, HPAD - HID)), (8, HPAD))
        ys = _sc_aggregate(xs, eidx, zhbm, dummy, dummy_i)
        s1, s2 = _stats(ys, ni2d, nn2d, bvec)
        gvec = jnp.broadcast_to(jnp.pad(gammas[l], (0, HPAD - HID)), (8, HPAD))
        bevec = jnp.broadcast_to(jnp.pad(betas[l], (0, HPAD - HID)), (8, HPAD))
        if l < L - 1:
            h, *xs = _apply_mm(ys, ni2d, nn2d, bvec, h_in, s1, s2,
                               gvec, bevec, wps[l + 1], no2d)
        else:
            sums, cnt = _apply_readout(ys, ni2d, nn2d, bvec, h_in,
                                       s1, s2, gvec, bevec, gid2d)
    w1p = _pad2(W1, HPAD, 128)
    b1p = jnp.broadcast_to(jnp.pad(b1, (0, 128 - b1.shape[0])), (8, 128))
    w2p = _pad2(W2, 128, 128)
    b2p = jnp.broadcast_to(jnp.pad(b2, (0, 128 - b2.shape[0])), (8, 128))
    w3p = _pad2(W3, 128, 128)
    b3p = jnp.broadcast_to(jnp.pad(b3, (0, 128 - b3.shape[0])), (8, 128))
    out = _mlp(sums, cnt, w1p, b1p, w2p, b2p, w3p, b3p)
    return out[:, :NCLS]


# speedup vs baseline: 1.0118x; 1.0118x over previous
import functools

import jax
import jax.numpy as jnp
from jax import lax
from jax.experimental import pallas as pl
from jax.experimental.pallas import tpu as pltpu
from jax.experimental.pallas import tpu_sc as plsc

N = 100000
E = 1600000
G = 128
IN_DIM = 32
HID = 146
HPAD = 160
NCHUNK = HPAD // 16
NCLS = 10
L = 4
NBLK = 1000

NSUB = 16
EPT = 101376
E_PAD = EPT * NSUB
ROWS_PER_TILE = EPT // 128
SB = 3
NWAY = 4
NTRIP = ROWS_PER_TILE // (NWAY * SB)
NPAD_SH = 100096
SH_PER_TILE = NPAD_SH // NSUB
ZROWS = 391
NZCOPY = SH_PER_TILE // ZROWS

_sc_mesh = plsc.VectorSubcoreMesh(core_axis_name="c", subcore_axis_name="s")
_sc_params = pltpu.CompilerParams(use_tc_tiling_on_sc=False)


def _zero_fill(ref, nrows):
    def body(i, _):
        ref[i] = jnp.zeros((16,), jnp.float32)
        return 0

    lax.fori_loop(0, nrows, body, 0)


def _sc_degree_body(srcm, dstm, out_o, out_i, cnt_sh, ones_v, i1, zbuf):
    cid = lax.axis_index("c")
    sid = lax.axis_index("s")
    _zero_fill(zbuf, ZROWS)

    def fill_ones(i, _):
        ones_v[i] = jnp.ones((16,), jnp.float32)
        return 0

    lax.fori_loop(0, 128, fill_ones, 0)

    z0 = sid * SH_PER_TILE

    def zc(t, _):
        pltpu.sync_copy(zbuf, cnt_sh.at[pl.ds(z0 + t * ZROWS, ZROWS)])
        return 0

    lax.fori_loop(0, NZCOPY, zc, 0)
    plsc.subcore_barrier()

    row0 = sid * ROWS_PER_TILE

    for half in range(2):
        @pl.when(cid == half)
        def _():
            idxm = srcm if half == 0 else dstm

            def body(r, _):
                pltpu.sync_copy(idxm.at[pl.ds(row0 + r, 1)], i1)
                pltpu.sync_copy(ones_v, cnt_sh.at[i1.at[0]], add=True)
                return 0

            lax.fori_loop(0, ROWS_PER_TILE, body, 0)

    plsc.subcore_barrier()
    for half in range(2):
        @pl.when(cid == half)
        def _():
            out = out_o if half == 0 else out_i
            pltpu.sync_copy(cnt_sh.at[pl.ds(z0, SH_PER_TILE)],
                            out.at[pl.ds(z0, SH_PER_TILE)])


def _sc_degrees(srcm_deg, dstm):
    return pl.kernel(
        _sc_degree_body,
        out_type=[
            jax.ShapeDtypeStruct((NPAD_SH, 16), jnp.float32),
            jax.ShapeDtypeStruct((NPAD_SH, 16), jnp.float32),
        ],
        mesh=_sc_mesh,
        compiler_params=_sc_params,
        scratch_types=[
            pltpu.VMEM_SHARED((NPAD_SH, 16), jnp.float32),
            pltpu.VMEM((128, 16), jnp.float32),
            pltpu.VMEM((1, 128), jnp.int32),
            pltpu.VMEM((ZROWS, 16), jnp.float32),
        ],
    )(srcm_deg, dstm)


def _sc_agg_body(*refs):
    xs = refs[0:NCHUNK]
    eidx = refs[NCHUNK]
    zhbm = refs[NCHUNK + 1]
    dummy = refs[NCHUNK + 2]
    dummy_i = refs[NCHUNK + 3]
    ys = refs[NCHUNK + 4:2 * NCHUNK + 4]
    sc = refs[2 * NCHUNK + 4:]
    agg_sh = sc[0]
    rows = sc[1:1 + NWAY]
    idx = sc[1 + NWAY:1 + 2 * NWAY]
    semG = sc[1 + 2 * NWAY:1 + 3 * NWAY]
    semS = sc[1 + 3 * NWAY:1 + 4 * NWAY]
    semI = sc[1 + 4 * NWAY:1 + 5 * NWAY]

    cid = lax.axis_index("c")
    sid = lax.axis_index("s")
    z0 = sid * SH_PER_TILE
    row0 = sid * ROWS_PER_TILE

    for half in range(2):
        @pl.when(cid == half)
        def _():
            for c in range(half * 5, half * 5 + 5):
                table = xs[c]
                out = ys[c]
                pltpu.sync_copy(zhbm, agg_sh.at[pl.ds(z0, SH_PER_TILE)])
                plsc.subcore_barrier()

                for p in range(NWAY):
                    pltpu.sync_copy(eidx.at[pl.ds(row0 + p * SB, SB)], idx[p])

                def quad(bb, _):
                    gs = [None] * NWAY
                    for p in range(NWAY):
                        @pl.when(bb > 0)
                        def _wait_idx():
                            pltpu.make_async_copy(dummy_i, idx[p],
                                                  semI[p]).wait()
                        gs[p] = [
                            pltpu.async_copy(table.at[idx[p].at[j, 0]],
                                             rows[p].at[j], semG[p])
                            for j in range(SB)
                        ]
                    for p in range(NWAY):
                        for cp in gs[p]:
                            cp.wait()
                        for j in range(SB):
                            pltpu.async_copy(rows[p].at[j],
                                             agg_sh.at[idx[p].at[j, 1]],
                                             semS[p], add=True)
                    for p in range(NWAY):
                        pltpu.make_async_copy(dummy, rows[p], semS[p]).wait()

                        @pl.when(bb < NTRIP - 1)
                        def _prefetch():
                            base = row0 + ((bb + 1) * NWAY + p) * SB
                            pltpu.async_copy(eidx.at[pl.ds(base, SB)],
                                             idx[p], semI[p])
                    return 0

                lax.fori_loop(0, NTRIP, quad, 0)
                plsc.subcore_barrier()
                pltpu.sync_copy(agg_sh.at[pl.ds(z0, SH_PER_TILE)],
                                out.at[pl.ds(z0, SH_PER_TILE)])
                plsc.subcore_barrier()


def _sc_aggregate(xs, eidx, zhbm, dummy, dummy_i):
    return pl.kernel(
        _sc_agg_body,
        out_type=[jax.ShapeDtypeStruct((NPAD_SH, 16), jnp.float32)
                  for _ in range(NCHUNK)],
        mesh=_sc_mesh,
        compiler_params=_sc_params,
        scratch_types=(
            [pltpu.VMEM_SHARED((NPAD_SH, 16), jnp.float32)]
            + [pltpu.VMEM((SB, 128, 16), jnp.float32) for _ in range(NWAY)]
            + [pltpu.VMEM((SB, 2, 128), jnp.int32) for _ in range(NWAY)]
            + [pltpu.SemaphoreType.DMA for _ in range(3 * NWAY)]
        ),
    )(*xs, eidx, zhbm, dummy, dummy_i)


def _emb_body(nf_ref, w_ref, b_ref, w0_ref, no_ref, h_ref, *out_refs):
    h = (jnp.dot(nf_ref[...], w_ref[...],
                 preferred_element_type=jnp.float32)
         + b_ref[...][0:1, :])
    h_ref[...] = h
    acc = jnp.dot(h, w0_ref[...],
                  preferred_element_type=jnp.float32) * no_ref[...]
    for c in range(NCHUNK):
        out_refs[c][...] = acc[:, 16 * c:16 * (c + 1)]


def _emb_mm(nf, wp, bp, w0p, no2d):
    return pl.pallas_call(
        _emb_body,
        grid=(N // NBLK,),
        in_specs=[
            pl.BlockSpec((NBLK, IN_DIM), lambda i: (i, 0)),
            pl.BlockSpec((IN_DIM, HPAD), lambda i: (0, 0)),
            pl.BlockSpec((8, HPAD), lambda i: (0, 0)),
            pl.BlockSpec((HPAD, HPAD), lambda i: (0, 0)),
            pl.BlockSpec((NBLK, 1), lambda i: (i, 0)),
        ],
        out_specs=([pl.BlockSpec((NBLK, HPAD), lambda i: (i, 0))]
                   + [pl.BlockSpec((NBLK, 16), lambda i: (i, 0))
                      for _ in range(NCHUNK)]),
        out_shape=([jax.ShapeDtypeStruct((N, HPAD), jnp.float32)]
                   + [jax.ShapeDtypeStruct((N, 16), jnp.float32)
                      for _ in range(NCHUNK)]),
    )(nf, wp, bp, w0p, no2d)


def _layer_mm_body(h_ref, w_ref, no_ref, *out_refs):
    acc = jnp.dot(h_ref[...], w_ref[...],
                  preferred_element_type=jnp.float32) * no_ref[...]
    for c in range(NCHUNK):
        out_refs[c][...] = acc[:, 16 * c:16 * (c + 1)]


def _layer_mm(h, wp, no2d):
    return pl.pallas_call(
        _layer_mm_body,
        grid=(N // NBLK,),
        in_specs=[
            pl.BlockSpec((NBLK, HPAD), lambda i: (i, 0)),
            pl.BlockSpec((HPAD, HPAD), lambda i: (0, 0)),
            pl.BlockSpec((NBLK, 1), lambda i: (i, 0)),
        ],
        out_specs=[pl.BlockSpec((NBLK, 16), lambda i: (i, 0))
                   for _ in range(NCHUNK)],
        out_shape=[jax.ShapeDtypeStruct((N, 16), jnp.float32)
                   for _ in range(NCHUNK)],
    )(h, wp, no2d)


def _stats_body(*refs):
    ys = refs[0:NCHUNK]
    ni_ref, nn_ref, b_ref = refs[NCHUNK:NCHUNK + 3]
    s1_ref, s2_ref = refs[NCHUNK + 3:]

    @pl.when(pl.program_id(0) == 0)
    def _init():
        s1_ref[...] = jnp.zeros_like(s1_ref)
        s2_ref[...] = jnp.zeros_like(s2_ref)

    ni = ni_ref[...]
    nn = nn_ref[...]
    for c in range(NCHUNK):
        h2 = (ys[c][...] * ni + b_ref[...][0:1, 16 * c:16 * (c + 1)]) * nn
        s1_ref[0:1, 16 * c:16 * (c + 1)] += jnp.sum(h2, axis=0, keepdims=True)
        s2_ref[0:1, 16 * c:16 * (c + 1)] += jnp.sum(h2 * h2, axis=0,
                                                    keepdims=True)


def _stats(ys, ni2d, nn2d, bvec):
    return pl.pallas_call(
        _stats_body,
        grid=(N // NBLK,),
        in_specs=(
            [pl.BlockSpec((NBLK, 16), lambda i: (i, 0))
             for _ in range(NCHUNK)]
            + [pl.BlockSpec((NBLK, 1), lambda i: (i, 0)),
               pl.BlockSpec((NBLK, 1), lambda i: (i, 0)),
               pl.BlockSpec((8, HPAD), lambda i: (0, 0))]
        ),
        out_specs=[pl.BlockSpec((8, HPAD), lambda i: (0, 0)),
                   pl.BlockSpec((8, HPAD), lambda i: (0, 0))],
        out_shape=[jax.ShapeDtypeStruct((8, HPAD), jnp.float32),
                   jax.ShapeDtypeStruct((8, HPAD), jnp.float32)],
    )(*ys, ni2d, nn2d, bvec)


def _apply_mm_body(*refs):
    ys = refs[0:NCHUNK]
    (ni_ref, nn_ref, b_ref, hin_ref, s1_ref, s2_ref, g_ref, be_ref,
     w_ref, no_ref) = refs[NCHUNK:NCHUNK + 10]
    hout_ref = refs[NCHUNK + 10]
    out_refs = refs[NCHUNK + 11:]
    ni = ni_ref[...]
    nn = nn_ref[...]
    mu = s1_ref[...] * (1.0 / N)
    var = s2_ref[...] * (1.0 / N) - mu * mu
    scv = jax.lax.rsqrt(var + 1e-5) * g_ref[...]
    shv = be_ref[...] - mu * scv
    cols = []
    for c in range(NCHUNK):
        sl = slice(16 * c, 16 * (c + 1))
        h2 = (ys[c][...] * ni + b_ref[...][0:1, sl]) * nn
        v = h2 * scv[0:1, sl] + shv[0:1, sl]
        cols.append(hin_ref[...][:, sl] + jnp.maximum(v, 0.0))
    h = jnp.concatenate(cols, axis=1)
    hout_ref[...] = h
    acc = jnp.dot(h, w_ref[...],
                  preferred_element_type=jnp.float32) * no_ref[...]
    for c in range(NCHUNK):
        out_refs[c][...] = acc[:, 16 * c:16 * (c + 1)]


def _apply_mm(ys, ni2d, nn2d, bvec, h_in, s1, s2, gvec, bevec, wp, no2d):
    return pl.pallas_call(
        _apply_mm_body,
        grid=(N // NBLK,),
        in_specs=(
            [pl.BlockSpec((NBLK, 16), lambda i: (i, 0))
             for _ in range(NCHUNK)]
            + [pl.BlockSpec((NBLK, 1), lambda i: (i, 0)),
               pl.BlockSpec((NBLK, 1), lambda i: (i, 0)),
               pl.BlockSpec((8, HPAD), lambda i: (0, 0)),
               pl.BlockSpec((NBLK, HPAD), lambda i: (i, 0)),
               pl.BlockSpec((8, HPAD), lambda i: (0, 0)),
               pl.BlockSpec((8, HPAD), lambda i: (0, 0)),
               pl.BlockSpec((8, HPAD), lambda i: (0, 0)),
               pl.BlockSpec((8, HPAD), lambda i: (0, 0)),
               pl.BlockSpec((HPAD, HPAD), lambda i: (0, 0)),
               pl.BlockSpec((NBLK, 1), lambda i: (i, 0))]
        ),
        out_specs=([pl.BlockSpec((NBLK, HPAD), lambda i: (i, 0))]
                   + [pl.BlockSpec((NBLK, 16), lambda i: (i, 0))
                      for _ in range(NCHUNK)]),
        out_shape=([jax.ShapeDtypeStruct((N, HPAD), jnp.float32)]
                   + [jax.ShapeDtypeStruct((N, 16), jnp.float32)
                      for _ in range(NCHUNK)]),
    )(*ys, ni2d, nn2d, bvec, h_in, s1, s2, gvec, bevec, wp, no2d)


def _apply_readout_body(*refs):
    ys = refs[0:NCHUNK]
    (ni_ref, nn_ref, b_ref, hin_ref, s1_ref, s2_ref, g_ref, be_ref,
     gid_ref) = refs[NCHUNK:NCHUNK + 9]
    sums_ref, cnt_ref = refs[NCHUNK + 9:]

    @pl.when(pl.program_id(0) == 0)
    def _init():
        sums_ref[...] = jnp.zeros_like(sums_ref)
        cnt_ref[...] = jnp.zeros_like(cnt_ref)

    ni = ni_ref[...]
    nn = nn_ref[...]
    mu = s1_ref[...] * (1.0 / N)
    var = s2_ref[...] * (1.0 / N) - mu * mu
    scv = jax.lax.rsqrt(var + 1e-5) * g_ref[...]
    shv = be_ref[...] - mu * scv
    cols = []
    for c in range(NCHUNK):
        sl = slice(16 * c, 16 * (c + 1))
        h2 = (ys[c][...] * ni + b_ref[...][0:1, sl]) * nn
        v = h2 * scv[0:1, sl] + shv[0:1, sl]
        cols.append(hin_ref[...][:, sl] + jnp.maximum(v, 0.0))
    h = jnp.concatenate(cols, axis=1)
    gid = gid_ref[...]
    onehot = (gid == jax.lax.broadcasted_iota(jnp.int32, (NBLK, G), 1)).astype(
        jnp.float32
    )
    sums_ref[...] += jnp.dot(onehot.T, h, preferred_element_type=jnp.float32)
    cnt_ref[...] += jnp.dot(
        onehot.T, jnp.ones((NBLK, 8), jnp.float32),
        preferred_element_type=jnp.float32
    )


def _apply_readout(ys, ni2d, nn2d, bvec, h_in, s1, s2, gvec, bevec, gid2d):
    return pl.pallas_call(
        _apply_readout_body,
        grid=(N // NBLK,),
        in_specs=(
            [pl.BlockSpec((NBLK, 16), lambda i: (i, 0))
             for _ in range(NCHUNK)]
            + [pl.BlockSpec((NBLK, 1), lambda i: (i, 0)),
               pl.BlockSpec((NBLK, 1), lambda i: (i, 0)),
               pl.BlockSpec((8, HPAD), lambda i: (0, 0)),
               pl.BlockSpec((NBLK, HPAD), lambda i: (i, 0)),
               pl.BlockSpec((8, HPAD), lambda i: (0, 0)),
               pl.BlockSpec((8, HPAD), lambda i: (0, 0)),
               pl.BlockSpec((8, HPAD), lambda i: (0, 0)),
               pl.BlockSpec((8, HPAD), lambda i: (0, 0)),
               pl.BlockSpec((NBLK, 1), lambda i: (i, 0))]
        ),
        out_specs=[pl.BlockSpec((G, HPAD), lambda i: (0, 0)),
                   pl.BlockSpec((G, 8), lambda i: (0, 0))],
        out_shape=[jax.ShapeDtypeStruct((G, HPAD), jnp.float32),
                   jax.ShapeDtypeStruct((G, 8), jnp.float32)],
    )(*ys, ni2d, nn2d, bvec, h_in, s1, s2, gvec, bevec, gid2d)


def _mlp_body(sums_ref, cnt_ref, w1_ref, b1_ref, w2_ref, b2_ref, w3_ref, b3_ref,
              out_ref):
    cnt = jnp.maximum(cnt_ref[...][:, 0:1], 1.0)
    hg = sums_ref[...] / cnt
    z = jnp.maximum(jnp.dot(hg, w1_ref[...], preferred_element_type=jnp.float32)
                    + b1_ref[...][0:1, :], 0.0)
    z = jnp.maximum(jnp.dot(z, w2_ref[...], preferred_element_type=jnp.float32)
                    + b2_ref[...][0:1, :], 0.0)
    out_ref[...] = (jnp.dot(z, w3_ref[...], preferred_element_type=jnp.float32)
                    + b3_ref[...][0:1, :])


def _mlp(sums, cnt, w1p, b1p, w2p, b2p, w3p, b3p):
    return pl.pallas_call(
        _mlp_body,
        out_shape=jax.ShapeDtypeStruct((G, 128), jnp.float32),
    )(sums, cnt, w1p, b1p, w2p, b2p, w3p, b3p)


def _pad2(a, r, c):
    return jnp.pad(a, ((0, r - a.shape[0]), (0, c - a.shape[1])))


def kernel(nodes_feat, edges_feat, nodes_num_norm_sqrt, edges_num_norm_sqrt,
           edge_index, graph_ids, emb_W, emb_b, Ws, bs, gammas, betas,
           W1, b1, W2, b2, W3, b3):
    src = edge_index[0]
    dst = edge_index[1]
    epad = E_PAD - E
    srcm_agg = jnp.concatenate(
        [src, jnp.zeros((epad,), jnp.int32)]).reshape(-1, 128)
    dstm_agg = jnp.concatenate(
        [dst, jnp.full((epad,), N, jnp.int32)]).reshape(-1, 128)
    srcm_deg = jnp.concatenate(
        [src, jnp.full((epad,), N, jnp.int32)]).reshape(-1, 128)
    dstm = jnp.concatenate(
        [dst, jnp.full((epad,), N, jnp.int32)]).reshape(-1, 128)

    zhbm = jnp.zeros((SH_PER_TILE, 16), jnp.float32)
    dummy = jnp.zeros((SB, 128, 16), jnp.float32)
    dummy_i = jnp.zeros((SB, 2, 128), jnp.int32)
    eidx = jnp.stack([srcm_agg, dstm_agg], axis=1)
    dcnt_o, dcnt_i = _sc_degrees(srcm_deg, dstm)
    no2d = jnp.clip(dcnt_o[:N, 0:1], 1.0, None) ** -0.5
    ni2d = jnp.clip(dcnt_i[:N, 0:1], 1.0, None) ** -0.5
    nn2d = nodes_num_norm_sqrt

    embWp = jnp.pad(emb_W, ((0, 0), (0, HPAD - HID)))
    embbp = jnp.broadcast_to(jnp.pad(emb_b, (0, HPAD - HID)), (8, HPAD))
    wps = [jnp.pad(Ws[l], ((0, HPAD - HID), (0, HPAD - HID)))
           for l in range(L)]
    gid2d = graph_ids.reshape(N, 1)
    h, *xs = _emb_mm(nodes_feat, embWp, embbp, wps[0], no2d)
    for l in range(L):
        h_in = h
        bvec = jnp.broadcast_to(jnp.pad(bs[l], (0, HPAD - HID)), (8, HPAD))
        ys = _sc_aggregate(xs, eidx, zhbm, dummy, dummy_i)
        s1, s2 = _stats(ys, ni2d, nn2d, bvec)
        gvec = jnp.broadcast_to(jnp.pad(gammas[l], (0, HPAD - HID)), (8, HPAD))
        bevec = jnp.broadcast_to(jnp.pad(betas[l], (0, HPAD - HID)), (8, HPAD))
        if l < L - 1:
            h, *xs = _apply_mm(ys, ni2d, nn2d, bvec, h_in, s1, s2,
                               gvec, bevec, wps[l + 1], no2d)
        else:
            sums, cnt = _apply_readout(ys, ni2d, nn2d, bvec, h_in,
                                       s1, s2, gvec, bevec, gid2d)
    w1p = _pad2(W1, HPAD, 128)
    b1p = jnp.broadcast_to(jnp.pad(b1, (0, 128 - b1.shape[0])), (8, 128))
    w2p = _pad2(W2, 128, 128)
    b2p = jnp.broadcast_to(jnp.pad(b2, (0, 128 - b2.shape[0])), (8, 128))
    w3p = _pad2(W3, 128, 128)
    b3p = jnp.broadcast_to(jnp.pad(b3, (0, 128 - b3.shape[0])), (8, 128))
    out = _mlp(sums, cnt, w1p, b1p, w2p, b2p, w3p, b3p)
    return out[:, :NCLS]


# speedup vs baseline: 1.0405x; 1.0284x over previous
import functools

import jax
import jax.numpy as jnp
from jax import lax
from jax.experimental import pallas as pl
from jax.experimental.pallas import tpu as pltpu
from jax.experimental.pallas import tpu_sc as plsc

N = 100000
E = 1600000
G = 128
IN_DIM = 32
HID = 146
HPAD = 160
NCHUNK = HPAD // 16
NCLS = 10
L = 4
NBLK = 1000

NSUB = 16
EPT = 101376
E_PAD = EPT * NSUB
ROWS_PER_TILE = EPT // 128
SB = 3
NWAY = 4
NTRIP = ROWS_PER_TILE // (NWAY * SB)
NPAD_SH = 100096
SH_PER_TILE = NPAD_SH // NSUB
ZROWS = 391
NZCOPY = SH_PER_TILE // ZROWS

_sc_mesh = plsc.VectorSubcoreMesh(core_axis_name="c", subcore_axis_name="s")
_sc_params = pltpu.CompilerParams(use_tc_tiling_on_sc=False)


def _zero_fill(ref, nrows):
    def body(i, _):
        ref[i] = jnp.zeros((16,), jnp.float32)
        return 0

    lax.fori_loop(0, nrows, body, 0)


def _sc_degree_body(srcm, dstm, out_o, out_i, cnt_sh, ones_v, i1, zbuf,
                    semD):
    cid = lax.axis_index("c")
    sid = lax.axis_index("s")
    _zero_fill(zbuf, ZROWS)

    def fill_ones(i, _):
        ones_v[i] = jnp.ones((16,), jnp.float32)
        return 0

    lax.fori_loop(0, 128, fill_ones, 0)

    z0 = sid * SH_PER_TILE

    def zc(t, _):
        pltpu.sync_copy(zbuf, cnt_sh.at[pl.ds(z0 + t * ZROWS, ZROWS)])
        return 0

    lax.fori_loop(0, NZCOPY, zc, 0)
    plsc.subcore_barrier()

    row0 = sid * ROWS_PER_TILE

    for half in range(2):
        @pl.when(cid == half)
        def _():
            idxm = srcm if half == 0 else dstm

            def body(r, _):
                pltpu.sync_copy(idxm.at[pl.ds(row0 + r * 8, 8)], i1)
                cps = [
                    pltpu.async_copy(ones_v, cnt_sh.at[i1.at[j]], semD,
                                     add=True)
                    for j in range(8)
                ]
                for cp in cps:
                    cp.wait()
                return 0

            lax.fori_loop(0, ROWS_PER_TILE // 8, body, 0)

    plsc.subcore_barrier()
    for half in range(2):
        @pl.when(cid == half)
        def _():
            out = out_o if half == 0 else out_i
            pltpu.sync_copy(cnt_sh.at[pl.ds(z0, SH_PER_TILE)],
                            out.at[pl.ds(z0, SH_PER_TILE)])


def _sc_degrees(srcm_deg, dstm):
    return pl.kernel(
        _sc_degree_body,
        out_type=[
            jax.ShapeDtypeStruct((NPAD_SH, 16), jnp.float32),
            jax.ShapeDtypeStruct((NPAD_SH, 16), jnp.float32),
        ],
        mesh=_sc_mesh,
        compiler_params=_sc_params,
        scratch_types=[
            pltpu.VMEM_SHARED((NPAD_SH, 16), jnp.float32),
            pltpu.VMEM((128, 16), jnp.float32),
            pltpu.VMEM((8, 128), jnp.int32),
            pltpu.VMEM((ZROWS, 16), jnp.float32),
            pltpu.SemaphoreType.DMA,
        ],
    )(srcm_deg, dstm)


def _sc_agg_body(*refs):
    xs = refs[0:NCHUNK]
    eidx = refs[NCHUNK]
    zhbm = refs[NCHUNK + 1]
    dummy = refs[NCHUNK + 2]
    dummy_i = refs[NCHUNK + 3]
    ys = refs[NCHUNK + 4:2 * NCHUNK + 4]
    sc = refs[2 * NCHUNK + 4:]
    agg_sh = sc[0]
    rows = sc[1:1 + NWAY]
    idx = sc[1 + NWAY:1 + 2 * NWAY]
    semG = sc[1 + 2 * NWAY:1 + 3 * NWAY]
    semS = sc[1 + 3 * NWAY:1 + 4 * NWAY]
    semI = sc[1 + 4 * NWAY:1 + 5 * NWAY]

    cid = lax.axis_index("c")
    sid = lax.axis_index("s")
    z0 = sid * SH_PER_TILE
    row0 = sid * ROWS_PER_TILE

    for half in range(2):
        @pl.when(cid == half)
        def _():
            for c in range(half * 5, half * 5 + 5):
                table = xs[c]
                out = ys[c]
                pltpu.sync_copy(zhbm, agg_sh.at[pl.ds(z0, SH_PER_TILE)])
                plsc.subcore_barrier()

                for p in range(NWAY):
                    pltpu.sync_copy(eidx.at[pl.ds(row0 + p * SB, SB)], idx[p])

                def quad(bb, _):
                    gs = [None] * NWAY
                    for p in range(NWAY):
                        @pl.when(bb > 0)
                        def _wait_idx():
                            pltpu.make_async_copy(dummy_i, idx[p],
                                                  semI[p]).wait()
                        gs[p] = [
                            pltpu.async_copy(table.at[idx[p].at[j, 0]],
                                             rows[p].at[j], semG[p])
                            for j in range(SB)
                        ]
                    for p in range(NWAY):
                        for cp in gs[p]:
                            cp.wait()
                        for j in range(SB):
                            pltpu.async_copy(rows[p].at[j],
                                             agg_sh.at[idx[p].at[j, 1]],
                                             semS[p], add=True)
                    for p in range(NWAY):
                        pltpu.make_async_copy(dummy, rows[p], semS[p]).wait()

                        @pl.when(bb < NTRIP - 1)
                        def _prefetch():
                            base = row0 + ((bb + 1) * NWAY + p) * SB
                            pltpu.async_copy(eidx.at[pl.ds(base, SB)],
                                             idx[p], semI[p])
                    return 0

                lax.fori_loop(0, NTRIP, quad, 0)
                plsc.subcore_barrier()
                pltpu.sync_copy(agg_sh.at[pl.ds(z0, SH_PER_TILE)],
                                out.at[pl.ds(z0, SH_PER_TILE)])
                plsc.subcore_barrier()


def _sc_aggregate(xs, eidx, zhbm, dummy, dummy_i):
    return pl.kernel(
        _sc_agg_body,
        out_type=[jax.ShapeDtypeStruct((NPAD_SH, 16), jnp.float32)
                  for _ in range(NCHUNK)],
        mesh=_sc_mesh,
        compiler_params=_sc_params,
        scratch_types=(
            [pltpu.VMEM_SHARED((NPAD_SH, 16), jnp.float32)]
            + [pltpu.VMEM((SB, 128, 16), jnp.float32) for _ in range(NWAY)]
            + [pltpu.VMEM((SB, 2, 128), jnp.int32) for _ in range(NWAY)]
            + [pltpu.SemaphoreType.DMA for _ in range(3 * NWAY)]
        ),
    )(*xs, eidx, zhbm, dummy, dummy_i)


def _emb_body(nf_ref, w_ref, b_ref, w0_ref, no_ref, h_ref, *out_refs):
    h = (jnp.dot(nf_ref[...], w_ref[...],
                 preferred_element_type=jnp.float32)
         + b_ref[...][0:1, :])
    h_ref[...] = h
    acc = jnp.dot(h, w0_ref[...],
                  preferred_element_type=jnp.float32) * no_ref[...]
    for c in range(NCHUNK):
        out_refs[c][...] = acc[:, 16 * c:16 * (c + 1)]


def _emb_mm(nf, wp, bp, w0p, no2d):
    return pl.pallas_call(
        _emb_body,
        grid=(N // NBLK,),
        in_specs=[
            pl.BlockSpec((NBLK, IN_DIM), lambda i: (i, 0)),
            pl.BlockSpec((IN_DIM, HPAD), lambda i: (0, 0)),
            pl.BlockSpec((8, HPAD), lambda i: (0, 0)),
            pl.BlockSpec((HPAD, HPAD), lambda i: (0, 0)),
            pl.BlockSpec((NBLK, 1), lambda i: (i, 0)),
        ],
        out_specs=([pl.BlockSpec((NBLK, HPAD), lambda i: (i, 0))]
                   + [pl.BlockSpec((NBLK, 16), lambda i: (i, 0))
                      for _ in range(NCHUNK)]),
        out_shape=([jax.ShapeDtypeStruct((N, HPAD), jnp.float32)]
                   + [jax.ShapeDtypeStruct((N, 16), jnp.float32)
                      for _ in range(NCHUNK)]),
    )(nf, wp, bp, w0p, no2d)


def _layer_mm_body(h_ref, w_ref, no_ref, *out_refs):
    acc = jnp.dot(h_ref[...], w_ref[...],
                  preferred_element_type=jnp.float32) * no_ref[...]
    for c in range(NCHUNK):
        out_refs[c][...] = acc[:, 16 * c:16 * (c + 1)]


def _layer_mm(h, wp, no2d):
    return pl.pallas_call(
        _layer_mm_body,
        grid=(N // NBLK,),
        in_specs=[
            pl.BlockSpec((NBLK, HPAD), lambda i: (i, 0)),
            pl.BlockSpec((HPAD, HPAD), lambda i: (0, 0)),
            pl.BlockSpec((NBLK, 1), lambda i: (i, 0)),
        ],
        out_specs=[pl.BlockSpec((NBLK, 16), lambda i: (i, 0))
                   for _ in range(NCHUNK)],
        out_shape=[jax.ShapeDtypeStruct((N, 16), jnp.float32)
                   for _ in range(NCHUNK)],
    )(h, wp, no2d)


def _stats_body(*refs):
    ys = refs[0:NCHUNK]
    ni_ref, nn_ref, b_ref = refs[NCHUNK:NCHUNK + 3]
    s1_ref, s2_ref = refs[NCHUNK + 3:]

    @pl.when(pl.program_id(0) == 0)
    def _init():
        s1_ref[...] = jnp.zeros_like(s1_ref)
        s2_ref[...] = jnp.zeros_like(s2_ref)

    ni = ni_ref[...]
    nn = nn_ref[...]
    for c in range(NCHUNK):
        h2 = (ys[c][...] * ni + b_ref[...][0:1, 16 * c:16 * (c + 1)]) * nn
        s1_ref[0:1, 16 * c:16 * (c + 1)] += jnp.sum(h2, axis=0, keepdims=True)
        s2_ref[0:1, 16 * c:16 * (c + 1)] += jnp.sum(h2 * h2, axis=0,
                                                    keepdims=True)


def _stats(ys, ni2d, nn2d, bvec):
    return pl.pallas_call(
        _stats_body,
        grid=(N // NBLK,),
        in_specs=(
            [pl.BlockSpec((NBLK, 16), lambda i: (i, 0))
             for _ in range(NCHUNK)]
            + [pl.BlockSpec((NBLK, 1), lambda i: (i, 0)),
               pl.BlockSpec((NBLK, 1), lambda i: (i, 0)),
               pl.BlockSpec((8, HPAD), lambda i: (0, 0))]
        ),
        out_specs=[pl.BlockSpec((8, HPAD), lambda i: (0, 0)),
                   pl.BlockSpec((8, HPAD), lambda i: (0, 0))],
        out_shape=[jax.ShapeDtypeStruct((8, HPAD), jnp.float32),
                   jax.ShapeDtypeStruct((8, HPAD), jnp.float32)],
    )(*ys, ni2d, nn2d, bvec)


def _apply_mm_body(*refs):
    ys = refs[0:NCHUNK]
    (ni_ref, nn_ref, b_ref, hin_ref, s1_ref, s2_ref, g_ref, be_ref,
     w_ref, no_ref) = refs[NCHUNK:NCHUNK + 10]
    hout_ref = refs[NCHUNK + 10]
    out_refs = refs[NCHUNK + 11:]
    ni = ni_ref[...]
    nn = nn_ref[...]
    mu = s1_ref[...] * (1.0 / N)
    var = s2_ref[...] * (1.0 / N) - mu * mu
    scv = jax.lax.rsqrt(var + 1e-5) * g_ref[...]
    shv = be_ref[...] - mu * scv
    cols = []
    for c in range(NCHUNK):
        sl = slice(16 * c, 16 * (c + 1))
        h2 = (ys[c][...] * ni + b_ref[...][0:1, sl]) * nn
        v = h2 * scv[0:1, sl] + shv[0:1, sl]
        cols.append(hin_ref[...][:, sl] + jnp.maximum(v, 0.0))
    h = jnp.concatenate(cols, axis=1)
    hout_ref[...] = h
    acc = jnp.dot(h, w_ref[...],
                  preferred_element_type=jnp.float32) * no_ref[...]
    for c in range(NCHUNK):
        out_refs[c][...] = acc[:, 16 * c:16 * (c + 1)]


def _apply_mm(ys, ni2d, nn2d, bvec, h_in, s1, s2, gvec, bevec, wp, no2d):
    return pl.pallas_call(
        _apply_mm_body,
        grid=(N // NBLK,),
        in_specs=(
            [pl.BlockSpec((NBLK, 16), lambda i: (i, 0))
             for _ in range(NCHUNK)]
            + [pl.BlockSpec((NBLK, 1), lambda i: (i, 0)),
               pl.BlockSpec((NBLK, 1), lambda i: (i, 0)),
               pl.BlockSpec((8, HPAD), lambda i: (0, 0)),
               pl.BlockSpec((NBLK, HPAD), lambda i: (i, 0)),
               pl.BlockSpec((8, HPAD), lambda i: (0, 0)),
               pl.BlockSpec((8, HPAD), lambda i: (0, 0)),
               pl.BlockSpec((8, HPAD), lambda i: (0, 0)),
               pl.BlockSpec((8, HPAD), lambda i: (0, 0)),
               pl.BlockSpec((HPAD, HPAD), lambda i: (0, 0)),
               pl.BlockSpec((NBLK, 1), lambda i: (i, 0))]
        ),
        out_specs=([pl.BlockSpec((NBLK, HPAD), lambda i: (i, 0))]
                   + [pl.BlockSpec((NBLK, 16), lambda i: (i, 0))
                      for _ in range(NCHUNK)]),
        out_shape=([jax.ShapeDtypeStruct((N, HPAD), jnp.float32)]
                   + [jax.ShapeDtypeStruct((N, 16), jnp.float32)
                      for _ in range(NCHUNK)]),
    )(*ys, ni2d, nn2d, bvec, h_in, s1, s2, gvec, bevec, wp, no2d)


def _apply_readout_body(*refs):
    ys = refs[0:NCHUNK]
    (ni_ref, nn_ref, b_ref, hin_ref, s1_ref, s2_ref, g_ref, be_ref,
     gid_ref) = refs[NCHUNK:NCHUNK + 9]
    sums_ref, cnt_ref = refs[NCHUNK + 9:]

    @pl.when(pl.program_id(0) == 0)
    def _init():
        sums_ref[...] = jnp.zeros_like(sums_ref)
        cnt_ref[...] = jnp.zeros_like(cnt_ref)

    ni = ni_ref[...]
    nn = nn_ref[...]
    mu = s1_ref[...] * (1.0 / N)
    var = s2_ref[...] * (1.0 / N) - mu * mu
    scv = jax.lax.rsqrt(var + 1e-5) * g_ref[...]
    shv = be_ref[...] - mu * scv
    cols = []
    for c in range(NCHUNK):
        sl = slice(16 * c, 16 * (c + 1))
        h2 = (ys[c][...] * ni + b_ref[...][0:1, sl]) * nn
        v = h2 * scv[0:1, sl] + shv[0:1, sl]
        cols.append(hin_ref[...][:, sl] + jnp.maximum(v, 0.0))
    h = jnp.concatenate(cols, axis=1)
    gid = gid_ref[...]
    onehot = (gid == jax.lax.broadcasted_iota(jnp.int32, (NBLK, G), 1)).astype(
        jnp.float32
    )
    sums_ref[...] += jnp.dot(onehot.T, h, preferred_element_type=jnp.float32)
    cnt_ref[...] += jnp.dot(
        onehot.T, jnp.ones((NBLK, 8), jnp.float32),
        preferred_element_type=jnp.float32
    )


def _apply_readout(ys, ni2d, nn2d, bvec, h_in, s1, s2, gvec, bevec, gid2d):
    return pl.pallas_call(
        _apply_readout_body,
        grid=(N // NBLK,),
        in_specs=(
            [pl.BlockSpec((NBLK, 16), lambda i: (i, 0))
             for _ in range(NCHUNK)]
            + [pl.BlockSpec((NBLK, 1), lambda i: (i, 0)),
               pl.BlockSpec((NBLK, 1), lambda i: (i, 0)),
               pl.BlockSpec((8, HPAD), lambda i: (0, 0)),
               pl.BlockSpec((NBLK, HPAD), lambda i: (i, 0)),
               pl.BlockSpec((8, HPAD), lambda i: (0, 0)),
               pl.BlockSpec((8, HPAD), lambda i: (0, 0)),
               pl.BlockSpec((8, HPAD), lambda i: (0, 0)),
               pl.BlockSpec((8, HPAD), lambda i: (0, 0)),
               pl.BlockSpec((NBLK, 1), lambda i: (i, 0))]
        ),
        out_specs=[pl.BlockSpec((G, HPAD), lambda i: (0, 0)),
                   pl.BlockSpec((G, 8), lambda i: (0, 0))],
        out_shape=[jax.ShapeDtypeStruct((G, HPAD), jnp.float32),
                   jax.ShapeDtypeStruct((G, 8), jnp.float32)],
    )(*ys, ni2d, nn2d, bvec, h_in, s1, s2, gvec, bevec, gid2d)


def _mlp_body(sums_ref, cnt_ref, w1_ref, b1_ref, w2_ref, b2_ref, w3_ref, b3_ref,
              out_ref):
    cnt = jnp.maximum(cnt_ref[...][:, 0:1], 1.0)
    hg = sums_ref[...] / cnt
    z = jnp.maximum(jnp.dot(hg, w1_ref[...], preferred_element_type=jnp.float32)
                    + b1_ref[...][0:1, :], 0.0)
    z = jnp.maximum(jnp.dot(z, w2_ref[...], preferred_element_type=jnp.float32)
                    + b2_ref[...][0:1, :], 0.0)
    out_ref[...] = (jnp.dot(z, w3_ref[...], preferred_element_type=jnp.float32)
                    + b3_ref[...][0:1, :])


def _mlp(sums, cnt, w1p, b1p, w2p, b2p, w3p, b3p):
    return pl.pallas_call(
        _mlp_body,
        out_shape=jax.ShapeDtypeStruct((G, 128), jnp.float32),
    )(sums, cnt, w1p, b1p, w2p, b2p, w3p, b3p)


def _pad2(a, r, c):
    return jnp.pad(a, ((0, r - a.shape[0]), (0, c - a.shape[1])))


def kernel(nodes_feat, edges_feat, nodes_num_norm_sqrt, edges_num_norm_sqrt,
           edge_index, graph_ids, emb_W, emb_b, Ws, bs, gammas, betas,
           W1, b1, W2, b2, W3, b3):
    src = edge_index[0]
    dst = edge_index[1]
    epad = E_PAD - E
    srcm_agg = jnp.concatenate(
        [src, jnp.zeros((epad,), jnp.int32)]).reshape(-1, 128)
    dstm_agg = jnp.concatenate(
        [dst, jnp.full((epad,), N, jnp.int32)]).reshape(-1, 128)
    srcm_deg = jnp.concatenate(
        [src, jnp.full((epad,), N, jnp.int32)]).reshape(-1, 128)
    dstm = jnp.concatenate(
        [dst, jnp.full((epad,), N, jnp.int32)]).reshape(-1, 128)

    zhbm = jnp.zeros((SH_PER_TILE, 16), jnp.float32)
    dummy = jnp.zeros((SB, 128, 16), jnp.float32)
    dummy_i = jnp.zeros((SB, 2, 128), jnp.int32)
    eidx = jnp.stack([srcm_agg, dstm_agg], axis=1)
    dcnt_o, dcnt_i = _sc_degrees(srcm_deg, dstm)
    no2d = jnp.clip(dcnt_o[:N, 0:1], 1.0, None) ** -0.5
    ni2d = jnp.clip(dcnt_i[:N, 0:1], 1.0, None) ** -0.5
    nn2d = nodes_num_norm_sqrt

    embWp = jnp.pad(emb_W, ((0, 0), (0, HPAD - HID)))
    embbp = jnp.broadcast_to(jnp.pad(emb_b, (0, HPAD - HID)), (8, HPAD))
    wps = [jnp.pad(Ws[l], ((0, HPAD - HID), (0, HPAD - HID)))
           for l in range(L)]
    gid2d = graph_ids.reshape(N, 1)
    h, *xs = _emb_mm(nodes_feat, embWp, embbp, wps[0], no2d)
    for l in range(L):
        h_in = h
        bvec = jnp.broadcast_to(jnp.pad(bs[l], (0, HPAD - HID)), (8, HPAD))
        ys = _sc_aggregate(xs, eidx, zhbm, dummy, dummy_i)
        s1, s2 = _stats(ys, ni2d, nn2d, bvec)
        gvec = jnp.broadcast_to(jnp.pad(gammas[l], (0, HPAD - HID)), (8, HPAD))
        bevec = jnp.broadcast_to(jnp.pad(betas[l], (0, HPAD - HID)), (8, HPAD))
        if l < L - 1:
            h, *xs = _apply_mm(ys, ni2d, nn2d, bvec, h_in, s1, s2,
                               gvec, bevec, wps[l + 1], no2d)
        else:
            sums, cnt = _apply_readout(ys, ni2d, nn2d, bvec, h_in,
                                       s1, s2, gvec, bevec, gid2d)
    w1p = _pad2(W1, HPAD, 128)
    b1p = jnp.broadcast_to(jnp.pad(b1, (0, 128 - b1.shape[0])), (8, 128))
    w2p = _pad2(W2, 128, 128)
    b2p = jnp.broadcast_to(jnp.pad(b2, (0, 128 - b2.shape[0])), (8, 128))
    w3p = _pad2(W3, 128, 128)
    b3p = jnp.broadcast_to(jnp.pad(b3, (0, 128 - b3.shape[0])), (8, 128))
    out = _mlp(sums, cnt, w1p, b1p, w2p, b2p, w3p, b3p)
    return out[:, :NCLS]


# speedup vs baseline: 1.0480x; 1.0072x over previous
import functools

import jax
import jax.numpy as jnp
from jax import lax
from jax.experimental import pallas as pl
from jax.experimental.pallas import tpu as pltpu
from jax.experimental.pallas import tpu_sc as plsc

N = 100000
E = 1600000
G = 128
IN_DIM = 32
HID = 146
HPAD = 160
NCHUNK = HPAD // 16
NCLS = 10
L = 4
NBLK = 1000

NSUB = 16
EPT = 101376
E_PAD = EPT * NSUB
ROWS_PER_TILE = EPT // 128
SB = 3
NWAY = 4
NTRIP = ROWS_PER_TILE // (NWAY * SB)
NPAD_SH = 100096
SH_PER_TILE = NPAD_SH // NSUB
ZROWS = 391
NZCOPY = SH_PER_TILE // ZROWS

_sc_mesh = plsc.VectorSubcoreMesh(core_axis_name="c", subcore_axis_name="s")
_sc_params = pltpu.CompilerParams(use_tc_tiling_on_sc=False)


def _zero_fill(ref, nrows):
    def body(i, _):
        ref[i] = jnp.zeros((16,), jnp.float32)
        return 0

    lax.fori_loop(0, nrows, body, 0)


def _sc_degree_body(srcm, dstm, out_o, out_i, cnt_sh, ones_v, i1, zbuf,
                    semD):
    cid = lax.axis_index("c")
    sid = lax.axis_index("s")
    _zero_fill(zbuf, ZROWS)

    def fill_ones(i, _):
        ones_v[i] = jnp.ones((16,), jnp.float32)
        return 0

    lax.fori_loop(0, 128, fill_ones, 0)

    z0 = sid * SH_PER_TILE

    def zc(t, _):
        pltpu.sync_copy(zbuf, cnt_sh.at[pl.ds(z0 + t * ZROWS, ZROWS)])
        return 0

    lax.fori_loop(0, NZCOPY, zc, 0)
    plsc.subcore_barrier()

    row0 = sid * ROWS_PER_TILE

    for half in range(2):
        @pl.when(cid == half)
        def _():
            idxm = srcm if half == 0 else dstm

            def body(r, _):
                pltpu.sync_copy(idxm.at[pl.ds(row0 + r * 8, 8)], i1)
                cps = [
                    pltpu.async_copy(ones_v, cnt_sh.at[i1.at[j]], semD,
                                     add=True)
                    for j in range(8)
                ]
                for cp in cps:
                    cp.wait()
                return 0

            lax.fori_loop(0, ROWS_PER_TILE // 8, body, 0)

    plsc.subcore_barrier()
    for half in range(2):
        @pl.when(cid == half)
        def _():
            out = out_o if half == 0 else out_i
            pltpu.sync_copy(cnt_sh.at[pl.ds(z0, SH_PER_TILE)],
                            out.at[pl.ds(z0, SH_PER_TILE)])


def _sc_degrees(srcm_deg, dstm):
    return pl.kernel(
        _sc_degree_body,
        out_type=[
            jax.ShapeDtypeStruct((NPAD_SH, 16), jnp.float32),
            jax.ShapeDtypeStruct((NPAD_SH, 16), jnp.float32),
        ],
        mesh=_sc_mesh,
        compiler_params=_sc_params,
        scratch_types=[
            pltpu.VMEM_SHARED((NPAD_SH, 16), jnp.float32),
            pltpu.VMEM((128, 16), jnp.float32),
            pltpu.VMEM((8, 128), jnp.int32),
            pltpu.VMEM((ZROWS, 16), jnp.float32),
            pltpu.SemaphoreType.DMA,
        ],
    )(srcm_deg, dstm)


def _sc_agg_body(*refs):
    xs = refs[0:NCHUNK]
    eidx = refs[NCHUNK]
    zhbm = refs[NCHUNK + 1]
    dummy = refs[NCHUNK + 2]
    dummy_i = refs[NCHUNK + 3]
    ys = refs[NCHUNK + 4:2 * NCHUNK + 4]
    sc = refs[2 * NCHUNK + 4:]
    agg_sh = sc[0]
    rows = sc[1:1 + NWAY]
    idx = sc[1 + NWAY:1 + 2 * NWAY]
    semG = sc[1 + 2 * NWAY:1 + 3 * NWAY]
    semS = sc[1 + 3 * NWAY:1 + 4 * NWAY]
    semI = sc[1 + 4 * NWAY:1 + 5 * NWAY]

    cid = lax.axis_index("c")
    sid = lax.axis_index("s")
    z0 = sid * SH_PER_TILE
    row0 = sid * ROWS_PER_TILE

    for half in range(2):
        @pl.when(cid == half)
        def _():
            for c in range(half * 5, half * 5 + 5):
                table = xs[c]
                out = ys[c]
                pltpu.sync_copy(zhbm, agg_sh.at[pl.ds(z0, SH_PER_TILE)])
                plsc.subcore_barrier()

                for p in range(NWAY):
                    pltpu.sync_copy(eidx.at[pl.ds(row0 + p * SB, SB)], idx[p])

                def quad(bb, _):
                    gs = [None] * NWAY
                    for p in range(NWAY):
                        @pl.when(bb > 0)
                        def _wait_idx():
                            pltpu.make_async_copy(dummy_i, idx[p],
                                                  semI[p]).wait()
                        gs[p] = [
                            pltpu.async_copy(table.at[idx[p].at[j, 0]],
                                             rows[p].at[j], semG[p])
                            for j in range(SB)
                        ]
                    for p in range(NWAY):
                        for cp in gs[p]:
                            cp.wait()
                        for j in range(SB):
                            pltpu.async_copy(rows[p].at[j],
                                             agg_sh.at[idx[p].at[j, 1]],
                                             semS[p], add=True)
                    for p in range(NWAY):
                        pltpu.make_async_copy(dummy, rows[p], semS[p]).wait()

                        @pl.when(bb < NTRIP - 1)
                        def _prefetch():
                            base = row0 + ((bb + 1) * NWAY + p) * SB
                            pltpu.async_copy(eidx.at[pl.ds(base, SB)],
                                             idx[p], semI[p])
                    return 0

                lax.fori_loop(0, NTRIP, quad, 0)
                plsc.subcore_barrier()
                pltpu.sync_copy(agg_sh.at[pl.ds(z0, SH_PER_TILE)],
                                out.at[pl.ds(z0, SH_PER_TILE)])
                plsc.subcore_barrier()


def _sc_aggregate(xs, eidx, zhbm, dummy, dummy_i):
    return pl.kernel(
        _sc_agg_body,
        out_type=[jax.ShapeDtypeStruct((NPAD_SH, 16), jnp.float32)
                  for _ in range(NCHUNK)],
        mesh=_sc_mesh,
        compiler_params=_sc_params,
        scratch_types=(
            [pltpu.VMEM_SHARED((NPAD_SH, 16), jnp.float32)]
            + [pltpu.VMEM((SB, 128, 16), jnp.float32) for _ in range(NWAY)]
            + [pltpu.VMEM((SB, 2, 128), jnp.int32) for _ in range(NWAY)]
            + [pltpu.SemaphoreType.DMA for _ in range(3 * NWAY)]
        ),
    )(*xs, eidx, zhbm, dummy, dummy_i)


def _emb_body(nf_ref, w_ref, b_ref, w0_ref, no_ref, h_ref, *out_refs):
    h = (jnp.dot(nf_ref[...], w_ref[...],
                 preferred_element_type=jnp.float32)
         + b_ref[...][0:1, :])
    h_ref[...] = h
    acc = jnp.dot(h, w0_ref[...],
                  preferred_element_type=jnp.float32) * no_ref[...]
    for c in range(NCHUNK):
        out_refs[c][...] = acc[:, 16 * c:16 * (c + 1)]


def _emb_mm(nf, wp, bp, w0p, no2d):
    return pl.pallas_call(
        _emb_body,
        grid=(N // NBLK,),
        in_specs=[
            pl.BlockSpec((NBLK, IN_DIM), lambda i: (i, 0)),
            pl.BlockSpec((IN_DIM, HPAD), lambda i: (0, 0)),
            pl.BlockSpec((8, HPAD), lambda i: (0, 0)),
            pl.BlockSpec((HPAD, HPAD), lambda i: (0, 0)),
            pl.BlockSpec((NBLK, 1), lambda i: (i, 0)),
        ],
        out_specs=([pl.BlockSpec((NBLK, HPAD), lambda i: (i, 0))]
                   + [pl.BlockSpec((NBLK, 16), lambda i: (i, 0))
                      for _ in range(NCHUNK)]),
        out_shape=([jax.ShapeDtypeStruct((N, HPAD), jnp.float32)]
                   + [jax.ShapeDtypeStruct((N, 16), jnp.float32)
                      for _ in range(NCHUNK)]),
    )(nf, wp, bp, w0p, no2d)


def _layer_mm_body(h_ref, w_ref, no_ref, *out_refs):
    acc = jnp.dot(h_ref[...], w_ref[...],
                  preferred_element_type=jnp.float32) * no_ref[...]
    for c in range(NCHUNK):
        out_refs[c][...] = acc[:, 16 * c:16 * (c + 1)]


def _layer_mm(h, wp, no2d):
    return pl.pallas_call(
        _layer_mm_body,
        grid=(N // NBLK,),
        in_specs=[
            pl.BlockSpec((NBLK, HPAD), lambda i: (i, 0)),
            pl.BlockSpec((HPAD, HPAD), lambda i: (0, 0)),
            pl.BlockSpec((NBLK, 1), lambda i: (i, 0)),
        ],
        out_specs=[pl.BlockSpec((NBLK, 16), lambda i: (i, 0))
                   for _ in range(NCHUNK)],
        out_shape=[jax.ShapeDtypeStruct((N, 16), jnp.float32)
                   for _ in range(NCHUNK)],
    )(h, wp, no2d)


def _stats_body(*refs):
    ys = refs[0:NCHUNK]
    ni_ref, nn_ref, b_ref = refs[NCHUNK:NCHUNK + 3]
    s1_ref, s2_ref = refs[NCHUNK + 3:]

    @pl.when(pl.program_id(0) == 0)
    def _init():
        s1_ref[...] = jnp.zeros_like(s1_ref)
        s2_ref[...] = jnp.zeros_like(s2_ref)

    ni = ni_ref[...]
    nn = nn_ref[...]
    for c in range(NCHUNK):
        h2 = (ys[c][...] * ni + b_ref[...][0:1, 16 * c:16 * (c + 1)]) * nn
        s1_ref[0:1, 16 * c:16 * (c + 1)] += jnp.sum(h2, axis=0, keepdims=True)
        s2_ref[0:1, 16 * c:16 * (c + 1)] += jnp.sum(h2 * h2, axis=0,
                                                    keepdims=True)


def _stats(ys, ni2d, nn2d, bvec):
    return pl.pallas_call(
        _stats_body,
        grid=(N // NBLK,),
        in_specs=(
            [pl.BlockSpec((NBLK, 16), lambda i: (i, 0))
             for _ in range(NCHUNK)]
            + [pl.BlockSpec((NBLK, 1), lambda i: (i, 0)),
               pl.BlockSpec((NBLK, 1), lambda i: (i, 0)),
               pl.BlockSpec((8, HPAD), lambda i: (0, 0))]
        ),
        out_specs=[pl.BlockSpec((8, HPAD), lambda i: (0, 0)),
                   pl.BlockSpec((8, HPAD), lambda i: (0, 0))],
        out_shape=[jax.ShapeDtypeStruct((8, HPAD), jnp.float32),
                   jax.ShapeDtypeStruct((8, HPAD), jnp.float32)],
    )(*ys, ni2d, nn2d, bvec)


def _apply_mm_body(*refs):
    ys = refs[0:NCHUNK]
    (ni_ref, nn_ref, b_ref, hin_ref, scv_ref, shv_ref,
     w_ref, no_ref) = refs[NCHUNK:NCHUNK + 8]
    hout_ref = refs[NCHUNK + 8]
    out_refs = refs[NCHUNK + 9:]
    ni = ni_ref[...]
    nn = nn_ref[...]
    scv = scv_ref[...]
    shv = shv_ref[...]
    cols = []
    for c in range(NCHUNK):
        sl = slice(16 * c, 16 * (c + 1))
        h2 = (ys[c][...] * ni + b_ref[...][0:1, sl]) * nn
        v = h2 * scv[0:1, sl] + shv[0:1, sl]
        cols.append(hin_ref[...][:, sl] + jnp.maximum(v, 0.0))
    h = jnp.concatenate(cols, axis=1)
    hout_ref[...] = h
    acc = jnp.dot(h, w_ref[...],
                  preferred_element_type=jnp.float32) * no_ref[...]
    for c in range(NCHUNK):
        out_refs[c][...] = acc[:, 16 * c:16 * (c + 1)]


def _apply_mm(ys, ni2d, nn2d, bvec, h_in, scale, shift, wp, no2d):
    return pl.pallas_call(
        _apply_mm_body,
        grid=(N // NBLK,),
        in_specs=(
            [pl.BlockSpec((NBLK, 16), lambda i: (i, 0))
             for _ in range(NCHUNK)]
            + [pl.BlockSpec((NBLK, 1), lambda i: (i, 0)),
               pl.BlockSpec((NBLK, 1), lambda i: (i, 0)),
               pl.BlockSpec((8, HPAD), lambda i: (0, 0)),
               pl.BlockSpec((NBLK, HPAD), lambda i: (i, 0)),
               pl.BlockSpec((8, HPAD), lambda i: (0, 0)),
               pl.BlockSpec((8, HPAD), lambda i: (0, 0)),
               pl.BlockSpec((HPAD, HPAD), lambda i: (0, 0)),
               pl.BlockSpec((NBLK, 1), lambda i: (i, 0))]
        ),
        out_specs=([pl.BlockSpec((NBLK, HPAD), lambda i: (i, 0))]
                   + [pl.BlockSpec((NBLK, 16), lambda i: (i, 0))
                      for _ in range(NCHUNK)]),
        out_shape=([jax.ShapeDtypeStruct((N, HPAD), jnp.float32)]
                   + [jax.ShapeDtypeStruct((N, 16), jnp.float32)
                      for _ in range(NCHUNK)]),
    )(*ys, ni2d, nn2d, bvec, h_in, scale, shift, wp, no2d)


def _apply_readout_body(*refs):
    ys = refs[0:NCHUNK]
    (ni_ref, nn_ref, b_ref, hin_ref, scv_ref, shv_ref,
     gid_ref) = refs[NCHUNK:NCHUNK + 7]
    sums_ref, cnt_ref = refs[NCHUNK + 7:]

    @pl.when(pl.program_id(0) == 0)
    def _init():
        sums_ref[...] = jnp.zeros_like(sums_ref)
        cnt_ref[...] = jnp.zeros_like(cnt_ref)

    ni = ni_ref[...]
    nn = nn_ref[...]
    scv = scv_ref[...]
    shv = shv_ref[...]
    cols = []
    for c in range(NCHUNK):
        sl = slice(16 * c, 16 * (c + 1))
        h2 = (ys[c][...] * ni + b_ref[...][0:1, sl]) * nn
        v = h2 * scv[0:1, sl] + shv[0:1, sl]
        cols.append(hin_ref[...][:, sl] + jnp.maximum(v, 0.0))
    h = jnp.concatenate(cols, axis=1)
    gid = gid_ref[...]
    onehot = (gid == jax.lax.broadcasted_iota(jnp.int32, (NBLK, G), 1)).astype(
        jnp.float32
    )
    sums_ref[...] += jnp.dot(onehot.T, h, preferred_element_type=jnp.float32)
    cnt_ref[...] += jnp.dot(
        onehot.T, jnp.ones((NBLK, 8), jnp.float32),
        preferred_element_type=jnp.float32
    )


def _apply_readout(ys, ni2d, nn2d, bvec, h_in, scale, shift, gid2d):
    return pl.pallas_call(
        _apply_readout_body,
        grid=(N // NBLK,),
        in_specs=(
            [pl.BlockSpec((NBLK, 16), lambda i: (i, 0))
             for _ in range(NCHUNK)]
            + [pl.BlockSpec((NBLK, 1), lambda i: (i, 0)),
               pl.BlockSpec((NBLK, 1), lambda i: (i, 0)),
               pl.BlockSpec((8, HPAD), lambda i: (0, 0)),
               pl.BlockSpec((NBLK, HPAD), lambda i: (i, 0)),
               pl.BlockSpec((8, HPAD), lambda i: (0, 0)),
               pl.BlockSpec((8, HPAD), lambda i: (0, 0)),
               pl.BlockSpec((NBLK, 1), lambda i: (i, 0))]
        ),
        out_specs=[pl.BlockSpec((G, HPAD), lambda i: (0, 0)),
                   pl.BlockSpec((G, 8), lambda i: (0, 0))],
        out_shape=[jax.ShapeDtypeStruct((G, HPAD), jnp.float32),
                   jax.ShapeDtypeStruct((G, 8), jnp.float32)],
    )(*ys, ni2d, nn2d, bvec, h_in, scale, shift, gid2d)


def _mlp_body(sums_ref, cnt_ref, w1_ref, b1_ref, w2_ref, b2_ref, w3_ref, b3_ref,
              out_ref):
    cnt = jnp.maximum(cnt_ref[...][:, 0:1], 1.0)
    hg = sums_ref[...] / cnt
    z = jnp.maximum(jnp.dot(hg, w1_ref[...], preferred_element_type=jnp.float32)
                    + b1_ref[...][0:1, :], 0.0)
    z = jnp.maximum(jnp.dot(z, w2_ref[...], preferred_element_type=jnp.float32)
                    + b2_ref[...][0:1, :], 0.0)
    out_ref[...] = (jnp.dot(z, w3_ref[...], preferred_element_type=jnp.float32)
                    + b3_ref[...][0:1, :])


def _mlp(sums, cnt, w1p, b1p, w2p, b2p, w3p, b3p):
    return pl.pallas_call(
        _mlp_body,
        out_shape=jax.ShapeDtypeStruct((G, 128), jnp.float32),
    )(sums, cnt, w1p, b1p, w2p, b2p, w3p, b3p)


def _pad2(a, r, c):
    return jnp.pad(a, ((0, r - a.shape[0]), (0, c - a.shape[1])))


def kernel(nodes_feat, edges_feat, nodes_num_norm_sqrt, edges_num_norm_sqrt,
           edge_index, graph_ids, emb_W, emb_b, Ws, bs, gammas, betas,
           W1, b1, W2, b2, W3, b3):
    src = edge_index[0]
    dst = edge_index[1]
    epad = E_PAD - E
    srcm_agg = jnp.concatenate(
        [src, jnp.zeros((epad,), jnp.int32)]).reshape(-1, 128)
    dstm_agg = jnp.concatenate(
        [dst, jnp.full((epad,), N, jnp.int32)]).reshape(-1, 128)
    srcm_deg = jnp.concatenate(
        [src, jnp.full((epad,), N, jnp.int32)]).reshape(-1, 128)
    dstm = jnp.concatenate(
        [dst, jnp.full((epad,), N, jnp.int32)]).reshape(-1, 128)

    zhbm = jnp.zeros((SH_PER_TILE, 16), jnp.float32)
    dummy = jnp.zeros((SB, 128, 16), jnp.float32)
    dummy_i = jnp.zeros((SB, 2, 128), jnp.int32)
    eidx = jnp.stack([srcm_agg, dstm_agg], axis=1)
    dcnt_o, dcnt_i = _sc_degrees(srcm_deg, dstm)
    no2d = jnp.clip(dcnt_o[:N, 0:1], 1.0, None) ** -0.5
    ni2d = jnp.clip(dcnt_i[:N, 0:1], 1.0, None) ** -0.5
    nn2d = nodes_num_norm_sqrt

    embWp = jnp.pad(emb_W, ((0, 0), (0, HPAD - HID)))
    embbp = jnp.broadcast_to(jnp.pad(emb_b, (0, HPAD - HID)), (8, HPAD))
    wps = [jnp.pad(Ws[l], ((0, HPAD - HID), (0, HPAD - HID)))
           for l in range(L)]
    gid2d = graph_ids.reshape(N, 1)
    h, *xs = _emb_mm(nodes_feat, embWp, embbp, wps[0], no2d)
    for l in range(L):
        h_in = h
        bvec = jnp.broadcast_to(jnp.pad(bs[l], (0, HPAD - HID)), (8, HPAD))
        ys = _sc_aggregate(xs, eidx, zhbm, dummy, dummy_i)
        s1, s2 = _stats(ys, ni2d, nn2d, bvec)
        mu = s1[0:1] / N
        var = s2[0:1] / N - mu * mu
        rstd = jax.lax.rsqrt(var + 1e-5)
        gp = jnp.pad(gammas[l], (0, HPAD - HID))[None, :]
        bp = jnp.pad(betas[l], (0, HPAD - HID))[None, :]
        scale = jnp.broadcast_to(rstd * gp, (8, HPAD))
        shift = jnp.broadcast_to(bp - mu * rstd * gp, (8, HPAD))
        if l < L - 1:
            h, *xs = _apply_mm(ys, ni2d, nn2d, bvec, h_in, scale, shift,
                               wps[l + 1], no2d)
        else:
            sums, cnt = _apply_readout(ys, ni2d, nn2d, bvec, h_in,
                                       scale, shift, gid2d)
    w1p = _pad2(W1, HPAD, 128)
    b1p = jnp.broadcast_to(jnp.pad(b1, (0, 128 - b1.shape[0])), (8, 128))
    w2p = _pad2(W2, 128, 128)
    b2p = jnp.broadcast_to(jnp.pad(b2, (0, 128 - b2.shape[0])), (8, 128))
    w3p = _pad2(W3, 128, 128)
    b3p = jnp.broadcast_to(jnp.pad(b3, (0, 128 - b3.shape[0])), (8, 128))
    out = _mlp(sums, cnt, w1p, b1p, w2p, b2p, w3p, b3p)
    return out[:, :NCLS]
